# Initial kernel scaffold; baseline (speedup 1.0000x reference)
import jax
import jax.numpy as jnp
from jax.experimental import pallas as pl


def kernel(token_ids, bbox, tok_emb, type_emb, size_emb, dir_emb, pos1d, pos2d_x, pos2d_y, pos2d_h, pos2d_w, gamma, beta):
    raise NotImplementedError("write your pallas kernel here")


# speedup vs baseline: 1.6197x; 1.6197x over previous
import functools

import jax
import jax.numpy as jnp
from jax import lax
from jax.experimental import pallas as pl
from jax.experimental.pallas import tpu as pltpu
from jax.experimental.pallas import tpu_sc as plsc

_HID = 768
_NSL = _HID // 16
_NW = 32
_TPW = 256
_G = 8
_NCHUNK = _TPW // _G
_EPSILON = 1e-12


def _vrsqrt(z16):
    i = lax.bitcast_convert_type(z16, jnp.int32)
    i = jnp.int32(0x5F3759DF) - lax.shift_right_logical(i, 1)
    y = lax.bitcast_convert_type(i, jnp.float32)
    half = z16 * 0.5
    for _ in range(3):
        y = y * (1.5 - half * y * y)
    return y


def _body(tok_hbm, idx6_hbm, tokemb, pos1d, px, py, ph, pw, cgb_hbm,
          out_hbm,
          tokidx, xi1, yi1, xi2, yi2, dyi, dxi,
          buf, posc, obuf, cgb, insem, outsem):
    cid = lax.axis_index("c")
    sid = lax.axis_index("s")
    wid = sid * 2 + cid
    tbase = wid * _TPW
    sbase = (wid % 8) * _TPW

    pltpu.sync_copy(tok_hbm.at[pl.ds(tbase, _TPW)], tokidx)
    pltpu.sync_copy(idx6_hbm.at[0, pl.ds(tbase, _TPW)], xi1)
    pltpu.sync_copy(idx6_hbm.at[1, pl.ds(tbase, _TPW)], yi1)
    pltpu.sync_copy(idx6_hbm.at[2, pl.ds(tbase, _TPW)], xi2)
    pltpu.sync_copy(idx6_hbm.at[3, pl.ds(tbase, _TPW)], yi2)
    pltpu.sync_copy(idx6_hbm.at[4, pl.ds(tbase, _TPW)], dyi)
    pltpu.sync_copy(idx6_hbm.at[5, pl.ds(tbase, _TPW)], dxi)
    pltpu.sync_copy(cgb_hbm, cgb)

    def chunk(c, _):
        co = pl.ds(c * _G, _G)
        copies = [
            pltpu.async_copy(tokemb.at[tokidx.at[co]], buf.at[0], insem),
            pltpu.async_copy(px.at[xi1.at[co]], buf.at[1], insem),
            pltpu.async_copy(py.at[yi1.at[co]], buf.at[2], insem),
            pltpu.async_copy(px.at[xi2.at[co]], buf.at[3], insem),
            pltpu.async_copy(py.at[yi2.at[co]], buf.at[4], insem),
            pltpu.async_copy(ph.at[dyi.at[co]], buf.at[5], insem),
            pltpu.async_copy(pw.at[dxi.at[co]], buf.at[6], insem),
            pltpu.async_copy(pos1d.at[pl.ds(sbase + c * _G, _G)], posc,
                             insem),
        ]
        for d in copies:
            d.wait()
        for t in range(_G):
            def p_sum(j, carry, t=t):
                sv, qv = carry
                o = pl.ds(j * 16, 16)
                a = buf[0, t, o] + buf[1, t, o]
                a = a + buf[2, t, o]
                a = a + buf[3, t, o]
                a = a + buf[4, t, o]
                a = a + buf[5, t, o]
                a = a + buf[6, t, o]
                a = a + posc[t, o]
                a = a + cgb[0, o]
                obuf[t, o] = a
                return (sv + a, qv + a * a)

            z16 = jnp.zeros((16,), jnp.float32)
            sv, qv = lax.fori_loop(0, _NSL, p_sum, (z16, z16))
            s1 = sv[0]
            s2 = qv[0]
            for k in range(1, 16):
                s1 = s1 + sv[k]
                s2 = s2 + qv[k]
            u = s1 * (1.0 / _HID)
            var = s2 * (1.0 / _HID) - u * u
            r = _vrsqrt(jnp.full((16,), var + _EPSILON, jnp.float32))

            def p_norm(j, _, t=t, u=u, r=r):
                o = pl.ds(j * 16, 16)
                x = obuf[t, o]
                obuf[t, o] = (x - u) * r * cgb[1, o] + cgb[2, o]
                return 0

            lax.fori_loop(0, _NSL, p_norm, 0)
        pltpu.async_copy(obuf, out_hbm.at[pl.ds(tbase + c * _G, _G)],
                         outsem).wait()
        return 0

    lax.fori_loop(0, _NCHUNK, chunk, 0)


@jax.jit
def _emb_ln(tok_flat, idx6, tok_emb, pos1d, px, py, ph, pw, cgb):
    mesh = plsc.VectorSubcoreMesh(core_axis_name="c", subcore_axis_name="s")
    f = pl.kernel(
        _body,
        mesh=mesh,
        out_type=jax.ShapeDtypeStruct((_NW * _TPW, _HID), jnp.float32),
        scratch_types=[
            pltpu.VMEM((_TPW,), jnp.int32),
            pltpu.VMEM((_TPW,), jnp.int32),
            pltpu.VMEM((_TPW,), jnp.int32),
            pltpu.VMEM((_TPW,), jnp.int32),
            pltpu.VMEM((_TPW,), jnp.int32),
            pltpu.VMEM((_TPW,), jnp.int32),
            pltpu.VMEM((_TPW,), jnp.int32),
            pltpu.VMEM((7, _G, _HID), jnp.float32),
            pltpu.VMEM((_G, _HID), jnp.float32),
            pltpu.VMEM((_G, _HID), jnp.float32),
            pltpu.VMEM((3, _HID), jnp.float32),
            pltpu.SemaphoreType.DMA,
            pltpu.SemaphoreType.DMA,
        ],
    )
    return f(tok_flat, idx6, tok_emb, pos1d, px, py, ph, pw, cgb)


def kernel(token_ids, bbox, tok_emb, type_emb, size_emb, dir_emb, pos1d,
           pos2d_x, pos2d_y, pos2d_h, pos2d_w, gamma, beta):
    B, S = token_ids.shape
    tok_flat = token_ids.reshape(-1).astype(jnp.int32)
    bb = bbox.reshape(-1, 4).astype(jnp.int32)
    x1, y1, x2, y2 = bb[:, 0], bb[:, 1], bb[:, 2], bb[:, 3]
    idx6 = jnp.stack([x1, y1, x2, y2, y2 - y1, x2 - x1])
    const_row = type_emb[0] + size_emb[0] + dir_emb[0]
    cgb = jnp.stack([const_row, gamma, beta])
    out = _emb_ln(tok_flat, idx6, tok_emb, pos1d,
                  pos2d_x, pos2d_y, pos2d_h, pos2d_w, cgb)
    return out.reshape(B, S, _HID)
